# baseline (speedup 1.0000x reference)
import jax
import jax.numpy as jnp
from jax.experimental import pallas as pl
from jax.experimental.pallas import tpu as pltpu

_N = 207
_NP = 208
_B = 8
_R = _B * _NP
_U = 64
_C = 128
_SEQ = 12
_HOR = 12
_M = 5


def _mm(a, b):
    return jnp.dot(a, b, preferred_element_type=jnp.float32)


def _fism_body(inp_ref, adj_ref,
               eWg0, ebg0, eWc0, ebc0, eWg1, ebg1, eWc1, ebc1,
               dWg0, dbg0, dWc0, dbc0, dWg1, dbg1, dWc1, dbc1,
               wproj, bproj,
               out_ref, h0_ref, h1_ref, sa_ref, sb_ref,
               srcA_ref, srcB_ref, d1_ref):
    a = adj_ref[...]
    d1 = jnp.maximum(jnp.sum(a, axis=1, keepdims=True), 1e-8)
    sa_ref[...] = (a / d1).T
    at = a.T
    d2 = jnp.maximum(jnp.sum(at, axis=1, keepdims=True), 1e-8)
    sb_ref[...] = (at / d2).T

    def dconv(src_ref, w_ref, b_ref, width):
        def bstep(b, _):
            sl = pl.ds(b * _NP, _NP)
            x0b = src_ref[sl, :]
            sa = sa_ref[...]
            sb = sb_ref[...]
            x1a = _mm(sa, x0b)
            x2a = 2.0 * _mm(sa, x1a) - x0b
            x1b = _mm(sb, x0b)
            x2b = 2.0 * _mm(sb, x1b) - x0b
            accb = (_mm(x0b, w_ref[0]) + _mm(x1a, w_ref[1])
                    + _mm(x2a, w_ref[2]) + _mm(x1b, w_ref[3])
                    + _mm(x2b, w_ref[4]))
            d1_ref[sl, :width] = accb
            return 0
        jax.lax.fori_loop(0, _B, bstep, 0, unroll=2)
        return d1_ref[:, :width] + b_ref[...]

    def cell0(x, wg, bg, wc, bc):
        h = h0_ref[...]
        srcA_ref[:, 0:1] = x
        srcA_ref[:, 1:1 + _U] = h
        val = jax.nn.sigmoid(dconv(srcA_ref, wg, bg, 2 * _U))
        r = val[:, :_U]
        u = val[:, _U:]
        srcA_ref[:, 1:1 + _U] = r * h
        c = jnp.tanh(dconv(srcA_ref, wc, bc, _U))
        nh = u * h + (1.0 - u) * c
        h0_ref[...] = nh
        return nh

    def cell1(x, wg, bg, wc, bc):
        h = h1_ref[...]
        srcB_ref[:, :_U] = x
        srcB_ref[:, _U:] = h
        val = jax.nn.sigmoid(dconv(srcB_ref, wg, bg, 2 * _U))
        r = val[:, :_U]
        u = val[:, _U:]
        srcB_ref[:, _U:] = r * h
        c = jnp.tanh(dconv(srcB_ref, wc, bc, _U))
        nh = u * h + (1.0 - u) * c
        h1_ref[...] = nh
        return nh

    h0_ref[...] = jnp.zeros((_R, _U), jnp.float32)
    h1_ref[...] = jnp.zeros((_R, _U), jnp.float32)
    srcA_ref[:, 1 + _U:] = jnp.zeros((_R, _C - 1 - _U), jnp.float32)

    def enc_body(t, carry):
        nh0 = cell0(inp_ref[t], eWg0, ebg0, eWc0, ebc0)
        cell1(nh0, eWg1, ebg1, eWc1, ebc1)
        return carry

    jax.lax.fori_loop(0, _SEQ, enc_body, 0, unroll=False)

    def dec_body(t, dec_in):
        nh0 = cell0(dec_in, dWg0, dbg0, dWc0, dbc0)
        nh1 = cell1(nh0, dWg1, dbg1, dWc1, dbc1)
        proj = _mm(nh1, wproj[...]) + bproj[...]
        out_ref[t] = proj
        return proj

    jax.lax.fori_loop(0, _HOR, dec_body,
                      jnp.zeros((_R, 1), jnp.float32), unroll=False)


def _reorder_w(w, cin, out):
    w = w.reshape(cin, _M, out).transpose(1, 0, 2)
    return jnp.pad(w, ((0, 0), (0, _C - cin), (0, 0)))


def kernel(inputs, adj, params):
    inp = jnp.pad(inputs, ((0, 0), (0, 0), (0, _NP - _N)))
    inp = inp.reshape(_SEQ, _R, 1)
    adj_p = jnp.pad(adj, ((0, _NP - _N), (0, _NP - _N)))

    args = [inp, adj_p]
    for mdl in ("enc", "dec"):
        for l in range(2):
            cin = (1 if l == 0 else _U) + _U
            args.append(_reorder_w(params[f"{mdl}_Wg{l}"], cin, 2 * _U))
            args.append(params[f"{mdl}_bg{l}"].reshape(1, 2 * _U))
            args.append(_reorder_w(params[f"{mdl}_Wc{l}"], cin, _U))
            args.append(params[f"{mdl}_bc{l}"].reshape(1, _U))
    args.append(params["W_proj"])
    args.append(params["b_proj"].reshape(1, 1))

    out = pl.pallas_call(
        _fism_body,
        out_shape=jax.ShapeDtypeStruct((_HOR, _R, 1), jnp.float32),
        scratch_shapes=[pltpu.VMEM((_R, _U), jnp.float32),
                        pltpu.VMEM((_R, _U), jnp.float32),
                        pltpu.VMEM((_NP, _NP), jnp.float32),
                        pltpu.VMEM((_NP, _NP), jnp.float32),
                        pltpu.VMEM((_R, _C), jnp.float32),
                        pltpu.VMEM((_R, _C), jnp.float32),
                        pltpu.VMEM((_R, _C), jnp.float32)],
    )(*args)

    return out.reshape(_HOR, _B, _NP)[:, :, :_N]


# speedup vs baseline: 1.4787x; 1.4787x over previous
import jax
import jax.numpy as jnp
from jax.experimental import pallas as pl
from jax.experimental.pallas import tpu as pltpu

_N = 207
_NP = 208
_B = 8
_R = _B * _NP
_U = 64
_C = 128
_SEQ = 12
_HOR = 12
_M = 5


def _mm(a, b):
    return jnp.dot(a, b, preferred_element_type=jnp.float32)


def _fism_body(inp_ref, adj_ref,
               eWg0, ebg0, eWc0, ebc0, eWg1, ebg1, eWc1, ebc1,
               dWg0, dbg0, dWc0, dbc0, dWg1, dbg1, dWc1, dbc1,
               wproj, bproj,
               out_ref, h0_ref, h1_ref, sa_ref, sb_ref,
               srcA_ref, srcB_ref, d1a_ref, d2a_ref, d1b_ref, d2b_ref):
    a = adj_ref[...]
    d1 = jnp.maximum(jnp.sum(a, axis=1, keepdims=True), 1e-8)
    sa_ref[...] = (a / d1).T
    at = a.T
    d2 = jnp.maximum(jnp.sum(at, axis=1, keepdims=True), 1e-8)
    sb_ref[...] = (at / d2).T

    def dconv(src_ref, w_ref, b_ref, width):
        def bstep(b, _):
            sl = pl.ds(b * _NP, _NP)
            x0b = src_ref[sl, :]
            sa = sa_ref[...]
            sb = sb_ref[...]
            x1a = _mm(sa, x0b)
            x1b = _mm(sb, x0b)
            d1a_ref[sl, :] = x1a
            d1b_ref[sl, :] = x1b
            d2a_ref[sl, :] = 2.0 * _mm(sa, x1a) - x0b
            d2b_ref[sl, :] = 2.0 * _mm(sb, x1b) - x0b
            return 0
        jax.lax.fori_loop(0, _B, bstep, 0, unroll=4)
        acc = (_mm(src_ref[...], w_ref[0])
               + _mm(d1a_ref[...], w_ref[1]) + _mm(d2a_ref[...], w_ref[2])
               + _mm(d1b_ref[...], w_ref[3]) + _mm(d2b_ref[...], w_ref[4]))
        del width
        return acc + b_ref[...]

    def cell0(x, wg, bg, wc, bc):
        h = h0_ref[...]
        srcA_ref[:, 0:1] = x
        srcA_ref[:, 1:1 + _U] = h
        val = jax.nn.sigmoid(dconv(srcA_ref, wg, bg, 2 * _U))
        r = val[:, :_U]
        u = val[:, _U:]
        srcA_ref[:, 1:1 + _U] = r * h
        c = jnp.tanh(dconv(srcA_ref, wc, bc, _U))
        nh = u * h + (1.0 - u) * c
        h0_ref[...] = nh
        return nh

    def cell1(x, wg, bg, wc, bc):
        h = h1_ref[...]
        srcB_ref[:, :_U] = x
        srcB_ref[:, _U:] = h
        val = jax.nn.sigmoid(dconv(srcB_ref, wg, bg, 2 * _U))
        r = val[:, :_U]
        u = val[:, _U:]
        srcB_ref[:, _U:] = r * h
        c = jnp.tanh(dconv(srcB_ref, wc, bc, _U))
        nh = u * h + (1.0 - u) * c
        h1_ref[...] = nh
        return nh

    h0_ref[...] = jnp.zeros((_R, _U), jnp.float32)
    h1_ref[...] = jnp.zeros((_R, _U), jnp.float32)
    srcA_ref[:, 1 + _U:] = jnp.zeros((_R, _C - 1 - _U), jnp.float32)

    def enc_body(t, carry):
        nh0 = cell0(inp_ref[t], eWg0, ebg0, eWc0, ebc0)
        cell1(nh0, eWg1, ebg1, eWc1, ebc1)
        return carry

    jax.lax.fori_loop(0, _SEQ, enc_body, 0, unroll=False)

    def dec_body(t, dec_in):
        nh0 = cell0(dec_in, dWg0, dbg0, dWc0, dbc0)
        nh1 = cell1(nh0, dWg1, dbg1, dWc1, dbc1)
        proj = _mm(nh1, wproj[...]) + bproj[...]
        out_ref[t] = proj
        return proj

    jax.lax.fori_loop(0, _HOR, dec_body,
                      jnp.zeros((_R, 1), jnp.float32), unroll=False)


def _reorder_w(w, cin, out):
    w = w.reshape(cin, _M, out).transpose(1, 0, 2)
    return jnp.pad(w, ((0, 0), (0, _C - cin), (0, 0)))


def kernel(inputs, adj, params):
    inp = jnp.pad(inputs, ((0, 0), (0, 0), (0, _NP - _N)))
    inp = inp.reshape(_SEQ, _R, 1)
    adj_p = jnp.pad(adj, ((0, _NP - _N), (0, _NP - _N)))

    args = [inp, adj_p]
    for mdl in ("enc", "dec"):
        for l in range(2):
            cin = (1 if l == 0 else _U) + _U
            args.append(_reorder_w(params[f"{mdl}_Wg{l}"], cin, 2 * _U))
            args.append(params[f"{mdl}_bg{l}"].reshape(1, 2 * _U))
            args.append(_reorder_w(params[f"{mdl}_Wc{l}"], cin, _U))
            args.append(params[f"{mdl}_bc{l}"].reshape(1, _U))
    args.append(params["W_proj"])
    args.append(params["b_proj"].reshape(1, 1))

    out = pl.pallas_call(
        _fism_body,
        out_shape=jax.ShapeDtypeStruct((_HOR, _R, 1), jnp.float32),
        scratch_shapes=[pltpu.VMEM((_R, _U), jnp.float32),
                        pltpu.VMEM((_R, _U), jnp.float32),
                        pltpu.VMEM((_NP, _NP), jnp.float32),
                        pltpu.VMEM((_NP, _NP), jnp.float32),
                        pltpu.VMEM((_R, _C), jnp.float32),
                        pltpu.VMEM((_R, _C), jnp.float32),
                        pltpu.VMEM((_R, _C), jnp.float32),
                        pltpu.VMEM((_R, _C), jnp.float32),
                        pltpu.VMEM((_R, _C), jnp.float32),
                        pltpu.VMEM((_R, _C), jnp.float32)],
    )(*args)

    return out.reshape(_HOR, _B, _NP)[:, :, :_N]


# speedup vs baseline: 1.6183x; 1.0944x over previous
import jax
import jax.numpy as jnp
from jax.experimental import pallas as pl
from jax.experimental.pallas import tpu as pltpu

_N = 207
_NP = 208
_B = 8
_R = _B * _NP
_U = 64
_C = 128
_SEQ = 12
_HOR = 12
_M = 5


def _mm(a, b):
    return jnp.dot(a, b, preferred_element_type=jnp.float32)


def _fism_body(inp_ref, adj_ref,
               eWg0, ebg0, eWc0, ebc0, eWg1, ebg1, eWc1, ebc1,
               dWg0, dbg0, dWc0, dbc0, dWg1, dbg1, dWc1, dbc1,
               wproj, bproj,
               out_ref, h0_ref, h1_ref, sa_ref, sb_ref,
               srcA_ref, srcB_ref, d1a_ref, d2a_ref, d1b_ref, d2b_ref):
    a = adj_ref[...]
    d1 = jnp.maximum(jnp.sum(a, axis=1, keepdims=True), 1e-8)
    sa_ref[...] = (a / d1).T
    at = a.T
    d2 = jnp.maximum(jnp.sum(at, axis=1, keepdims=True), 1e-8)
    sb_ref[...] = (at / d2).T

    def dconv(src_ref, w_ref, b_ref, width):
        def bstep(b, _):
            sl = pl.ds(b * _NP, _NP)
            x0b = src_ref[sl, :]
            sa = sa_ref[...]
            sb = sb_ref[...]
            x1a = _mm(sa, x0b)
            x1b = _mm(sb, x0b)
            d1a_ref[sl, :] = x1a
            d1b_ref[sl, :] = x1b
            d2a_ref[sl, :] = 2.0 * _mm(sa, x1a) - x0b
            d2b_ref[sl, :] = 2.0 * _mm(sb, x1b) - x0b
            return 0
        jax.lax.fori_loop(0, _B, bstep, 0, unroll=4)
        acc = (_mm(src_ref[...], w_ref[0])
               + _mm(d1a_ref[...], w_ref[1]) + _mm(d2a_ref[...], w_ref[2])
               + _mm(d1b_ref[...], w_ref[3]) + _mm(d2b_ref[...], w_ref[4]))
        del width
        return acc + b_ref[...]

    def dconv_rh(src_ref, w_ref, b_ref, xw):
        def pstep(p, _):
            o1 = p * (2 * _NP)
            sl1 = pl.ds(o1, _NP)
            sl2 = pl.ds(o1 + _NP, _NP)
            sa = sa_ref[...]
            sb = sb_ref[...]
            rhs = jnp.concatenate(
                [src_ref[sl1, xw:xw + _U], src_ref[sl2, xw:xw + _U]], axis=1)
            x1a = _mm(sa, rhs)
            x1b = _mm(sb, rhs)
            x2a = 2.0 * _mm(sa, x1a) - rhs
            x2b = 2.0 * _mm(sb, x1b) - rhs
            d1a_ref[sl1, xw:xw + _U] = x1a[:, :_U]
            d1a_ref[sl2, xw:xw + _U] = x1a[:, _U:]
            d2a_ref[sl1, xw:xw + _U] = x2a[:, :_U]
            d2a_ref[sl2, xw:xw + _U] = x2a[:, _U:]
            d1b_ref[sl1, xw:xw + _U] = x1b[:, :_U]
            d1b_ref[sl2, xw:xw + _U] = x1b[:, _U:]
            d2b_ref[sl1, xw:xw + _U] = x2b[:, :_U]
            d2b_ref[sl2, xw:xw + _U] = x2b[:, _U:]
            return 0
        jax.lax.fori_loop(0, _B // 2, pstep, 0, unroll=2)
        acc = (_mm(src_ref[...], w_ref[0])
               + _mm(d1a_ref[...], w_ref[1]) + _mm(d2a_ref[...], w_ref[2])
               + _mm(d1b_ref[...], w_ref[3]) + _mm(d2b_ref[...], w_ref[4]))
        return acc + b_ref[...]

    def cell0(x, wg, bg, wc, bc):
        h = h0_ref[...]
        srcA_ref[:, 0:1] = x
        srcA_ref[:, 1:1 + _U] = h
        val = jax.nn.sigmoid(dconv(srcA_ref, wg, bg, 2 * _U))
        r = val[:, :_U]
        u = val[:, _U:]
        srcA_ref[:, 1:1 + _U] = r * h
        c = jnp.tanh(dconv_rh(srcA_ref, wc, bc, 1))
        nh = u * h + (1.0 - u) * c
        h0_ref[...] = nh
        return nh

    def cell1(x, wg, bg, wc, bc):
        h = h1_ref[...]
        srcB_ref[:, :_U] = x
        srcB_ref[:, _U:] = h
        val = jax.nn.sigmoid(dconv(srcB_ref, wg, bg, 2 * _U))
        r = val[:, :_U]
        u = val[:, _U:]
        srcB_ref[:, _U:] = r * h
        c = jnp.tanh(dconv_rh(srcB_ref, wc, bc, _U))
        nh = u * h + (1.0 - u) * c
        h1_ref[...] = nh
        return nh

    h0_ref[...] = jnp.zeros((_R, _U), jnp.float32)
    h1_ref[...] = jnp.zeros((_R, _U), jnp.float32)
    srcA_ref[:, 1 + _U:] = jnp.zeros((_R, _C - 1 - _U), jnp.float32)

    def enc_body(t, carry):
        nh0 = cell0(inp_ref[t], eWg0, ebg0, eWc0, ebc0)
        cell1(nh0, eWg1, ebg1, eWc1, ebc1)
        return carry

    jax.lax.fori_loop(0, _SEQ, enc_body, 0, unroll=False)

    def dec_body(t, dec_in):
        nh0 = cell0(dec_in, dWg0, dbg0, dWc0, dbc0)
        nh1 = cell1(nh0, dWg1, dbg1, dWc1, dbc1)
        proj = _mm(nh1, wproj[...]) + bproj[...]
        out_ref[t] = proj
        return proj

    jax.lax.fori_loop(0, _HOR, dec_body,
                      jnp.zeros((_R, 1), jnp.float32), unroll=False)


def _reorder_w(w, cin, out):
    w = w.reshape(cin, _M, out).transpose(1, 0, 2)
    return jnp.pad(w, ((0, 0), (0, _C - cin), (0, 0)))


def kernel(inputs, adj, params):
    inp = jnp.pad(inputs, ((0, 0), (0, 0), (0, _NP - _N)))
    inp = inp.reshape(_SEQ, _R, 1)
    adj_p = jnp.pad(adj, ((0, _NP - _N), (0, _NP - _N)))

    args = [inp, adj_p]
    for mdl in ("enc", "dec"):
        for l in range(2):
            cin = (1 if l == 0 else _U) + _U
            args.append(_reorder_w(params[f"{mdl}_Wg{l}"], cin, 2 * _U))
            args.append(params[f"{mdl}_bg{l}"].reshape(1, 2 * _U))
            args.append(_reorder_w(params[f"{mdl}_Wc{l}"], cin, _U))
            args.append(params[f"{mdl}_bc{l}"].reshape(1, _U))
    args.append(params["W_proj"])
    args.append(params["b_proj"].reshape(1, 1))

    out = pl.pallas_call(
        _fism_body,
        out_shape=jax.ShapeDtypeStruct((_HOR, _R, 1), jnp.float32),
        scratch_shapes=[pltpu.VMEM((_R, _U), jnp.float32),
                        pltpu.VMEM((_R, _U), jnp.float32),
                        pltpu.VMEM((_NP, _NP), jnp.float32),
                        pltpu.VMEM((_NP, _NP), jnp.float32),
                        pltpu.VMEM((_R, _C), jnp.float32),
                        pltpu.VMEM((_R, _C), jnp.float32),
                        pltpu.VMEM((_R, _C), jnp.float32),
                        pltpu.VMEM((_R, _C), jnp.float32),
                        pltpu.VMEM((_R, _C), jnp.float32),
                        pltpu.VMEM((_R, _C), jnp.float32)],
    )(*args)

    return out.reshape(_HOR, _B, _NP)[:, :, :_N]


# speedup vs baseline: 2.0287x; 1.2536x over previous
import jax
import jax.numpy as jnp
from jax.experimental import pallas as pl
from jax.experimental.pallas import tpu as pltpu

_N = 207
_NP = 208
_B = 8
_R = _B * _NP
_U = 64
_C = 128
_SEQ = 12
_HOR = 12
_M = 5


def _mm(a, b):
    return jnp.dot(a, b, preferred_element_type=jnp.float32)


def _fism_body(inp_ref, adj_ref,
               eWg0, ebg0, eWc0, ebc0, eWg1, ebg1, eWc1, ebc1,
               dWg0, dbg0, dWc0, dbc0, dWg1, dbg1, dWc1, dbc1,
               wproj, bproj,
               out_ref, h0_ref, h1_ref, sa_ref, sb_ref, ta_ref, tb_ref,
               srcA_ref, srcB_ref, d1a_ref, d2a_ref, d1b_ref, d2b_ref):
    a = adj_ref[...]
    d1 = jnp.maximum(jnp.sum(a, axis=1, keepdims=True), 1e-8)
    sa_ref[...] = (a / d1).T
    at = a.T
    d2 = jnp.maximum(jnp.sum(at, axis=1, keepdims=True), 1e-8)
    sb_ref[...] = (at / d2).T
    eye = (jax.lax.broadcasted_iota(jnp.int32, (_NP, _NP), 0)
           == jax.lax.broadcasted_iota(jnp.int32, (_NP, _NP), 1)
           ).astype(jnp.float32)
    ta_ref[...] = 2.0 * _mm(sa_ref[...], sa_ref[...]) - eye
    tb_ref[...] = 2.0 * _mm(sb_ref[...], sb_ref[...]) - eye

    def dconv(src_ref, w_ref, b_ref, width):
        def bstep(b, _):
            sl = pl.ds(b * _NP, _NP)
            x0b = src_ref[sl, :]
            d1a_ref[sl, :] = _mm(sa_ref[...], x0b)
            d1b_ref[sl, :] = _mm(sb_ref[...], x0b)
            d2a_ref[sl, :] = _mm(ta_ref[...], x0b)
            d2b_ref[sl, :] = _mm(tb_ref[...], x0b)
            return 0
        jax.lax.fori_loop(0, _B, bstep, 0, unroll=4)
        acc = (_mm(src_ref[...], w_ref[0])
               + _mm(d1a_ref[...], w_ref[1]) + _mm(d2a_ref[...], w_ref[2])
               + _mm(d1b_ref[...], w_ref[3]) + _mm(d2b_ref[...], w_ref[4]))
        del width
        return acc + b_ref[...]

    def dconv_rh(src_ref, w_ref, b_ref, xw):
        def pstep(p, _):
            o1 = p * (2 * _NP)
            sl1 = pl.ds(o1, _NP)
            sl2 = pl.ds(o1 + _NP, _NP)
            rhs = jnp.concatenate(
                [src_ref[sl1, xw:xw + _U], src_ref[sl2, xw:xw + _U]], axis=1)
            x1a = _mm(sa_ref[...], rhs)
            x1b = _mm(sb_ref[...], rhs)
            x2a = _mm(ta_ref[...], rhs)
            x2b = _mm(tb_ref[...], rhs)
            d1a_ref[sl1, xw:xw + _U] = x1a[:, :_U]
            d1a_ref[sl2, xw:xw + _U] = x1a[:, _U:]
            d2a_ref[sl1, xw:xw + _U] = x2a[:, :_U]
            d2a_ref[sl2, xw:xw + _U] = x2a[:, _U:]
            d1b_ref[sl1, xw:xw + _U] = x1b[:, :_U]
            d1b_ref[sl2, xw:xw + _U] = x1b[:, _U:]
            d2b_ref[sl1, xw:xw + _U] = x2b[:, :_U]
            d2b_ref[sl2, xw:xw + _U] = x2b[:, _U:]
            return 0
        jax.lax.fori_loop(0, _B // 2, pstep, 0, unroll=2)
        acc = (_mm(src_ref[...], w_ref[0])
               + _mm(d1a_ref[...], w_ref[1]) + _mm(d2a_ref[...], w_ref[2])
               + _mm(d1b_ref[...], w_ref[3]) + _mm(d2b_ref[...], w_ref[4]))
        return acc + b_ref[...]

    def cell0(x, wg, bg, wc, bc):
        h = h0_ref[...]
        srcA_ref[:, 0:1] = x
        srcA_ref[:, 1:1 + _U] = h
        val = jax.nn.sigmoid(dconv(srcA_ref, wg, bg, 2 * _U))
        r = val[:, :_U]
        u = val[:, _U:]
        srcA_ref[:, 1:1 + _U] = r * h
        c = jnp.tanh(dconv_rh(srcA_ref, wc, bc, 1))
        nh = u * h + (1.0 - u) * c
        h0_ref[...] = nh
        return nh

    def cell1(x, wg, bg, wc, bc):
        h = h1_ref[...]
        srcB_ref[:, :_U] = x
        srcB_ref[:, _U:] = h
        val = jax.nn.sigmoid(dconv(srcB_ref, wg, bg, 2 * _U))
        r = val[:, :_U]
        u = val[:, _U:]
        srcB_ref[:, _U:] = r * h
        c = jnp.tanh(dconv_rh(srcB_ref, wc, bc, _U))
        nh = u * h + (1.0 - u) * c
        h1_ref[...] = nh
        return nh

    h0_ref[...] = jnp.zeros((_R, _U), jnp.float32)
    h1_ref[...] = jnp.zeros((_R, _U), jnp.float32)
    srcA_ref[:, 1 + _U:] = jnp.zeros((_R, _C - 1 - _U), jnp.float32)

    def enc_body(t, carry):
        nh0 = cell0(inp_ref[t], eWg0, ebg0, eWc0, ebc0)
        cell1(nh0, eWg1, ebg1, eWc1, ebc1)
        return carry

    jax.lax.fori_loop(0, _SEQ, enc_body, 0, unroll=False)

    def dec_body(t, dec_in):
        nh0 = cell0(dec_in, dWg0, dbg0, dWc0, dbc0)
        nh1 = cell1(nh0, dWg1, dbg1, dWc1, dbc1)
        proj = _mm(nh1, wproj[...]) + bproj[...]
        out_ref[t] = proj
        return proj

    jax.lax.fori_loop(0, _HOR, dec_body,
                      jnp.zeros((_R, 1), jnp.float32), unroll=False)


def _reorder_w(w, cin, out):
    w = w.reshape(cin, _M, out).transpose(1, 0, 2)
    return jnp.pad(w, ((0, 0), (0, _C - cin), (0, 0)))


def kernel(inputs, adj, params):
    inp = jnp.pad(inputs, ((0, 0), (0, 0), (0, _NP - _N)))
    inp = inp.reshape(_SEQ, _R, 1)
    adj_p = jnp.pad(adj, ((0, _NP - _N), (0, _NP - _N)))

    args = [inp, adj_p]
    for mdl in ("enc", "dec"):
        for l in range(2):
            cin = (1 if l == 0 else _U) + _U
            args.append(_reorder_w(params[f"{mdl}_Wg{l}"], cin, 2 * _U))
            args.append(params[f"{mdl}_bg{l}"].reshape(1, 2 * _U))
            args.append(_reorder_w(params[f"{mdl}_Wc{l}"], cin, _U))
            args.append(params[f"{mdl}_bc{l}"].reshape(1, _U))
    args.append(params["W_proj"])
    args.append(params["b_proj"].reshape(1, 1))

    out = pl.pallas_call(
        _fism_body,
        out_shape=jax.ShapeDtypeStruct((_HOR, _R, 1), jnp.float32),
        scratch_shapes=[pltpu.VMEM((_R, _U), jnp.float32),
                        pltpu.VMEM((_R, _U), jnp.float32),
                        pltpu.VMEM((_NP, _NP), jnp.float32),
                        pltpu.VMEM((_NP, _NP), jnp.float32),
                        pltpu.VMEM((_NP, _NP), jnp.float32),
                        pltpu.VMEM((_NP, _NP), jnp.float32),
                        pltpu.VMEM((_R, _C), jnp.float32),
                        pltpu.VMEM((_R, _C), jnp.float32),
                        pltpu.VMEM((_R, _C), jnp.float32),
                        pltpu.VMEM((_R, _C), jnp.float32),
                        pltpu.VMEM((_R, _C), jnp.float32),
                        pltpu.VMEM((_R, _C), jnp.float32)],
    )(*args)

    return out.reshape(_HOR, _B, _NP)[:, :, :_N]


# speedup vs baseline: 2.0325x; 1.0019x over previous
import jax
import jax.numpy as jnp
from jax.experimental import pallas as pl
from jax.experimental.pallas import tpu as pltpu

_N = 207
_NP = 208
_B = 8
_R = _B * _NP
_U = 64
_C = 128
_SEQ = 12
_HOR = 12
_M = 5


def _mm(a, b):
    return jnp.dot(a, b, preferred_element_type=jnp.float32)


def _fism_body(inp_ref, adj_ref,
               eWg0, ebg0, eWc0, ebc0, eWg1, ebg1, eWc1, ebc1,
               dWg0, dbg0, dWc0, dbc0, dWg1, dbg1, dWc1, dbc1,
               wproj, bproj,
               out_ref, h0_ref, h1_ref, sa_ref, sb_ref, ta_ref, tb_ref,
               srcA_ref, srcB_ref, d1a_ref, d2a_ref, d1b_ref, d2b_ref):
    a = adj_ref[...]
    d1 = jnp.maximum(jnp.sum(a, axis=1, keepdims=True), 1e-8)
    sa_ref[...] = (a / d1).T
    at = a.T
    d2 = jnp.maximum(jnp.sum(at, axis=1, keepdims=True), 1e-8)
    sb_ref[...] = (at / d2).T
    eye = (jax.lax.broadcasted_iota(jnp.int32, (_NP, _NP), 0)
           == jax.lax.broadcasted_iota(jnp.int32, (_NP, _NP), 1)
           ).astype(jnp.float32)
    def _mm_hi(x, y):
        return jnp.dot(x, y, preferred_element_type=jnp.float32,
                       precision=jax.lax.Precision.HIGHEST)
    ta_ref[...] = 2.0 * _mm_hi(sa_ref[...], sa_ref[...]) - eye
    tb_ref[...] = 2.0 * _mm_hi(sb_ref[...], sb_ref[...]) - eye

    def dconv(src_ref, w_ref, b_ref, width):
        def bstep(b, _):
            sl = pl.ds(b * _NP, _NP)
            x0b = src_ref[sl, :]
            d1a_ref[sl, :] = _mm(sa_ref[...], x0b)
            d1b_ref[sl, :] = _mm(sb_ref[...], x0b)
            d2a_ref[sl, :] = _mm(ta_ref[...], x0b)
            d2b_ref[sl, :] = _mm(tb_ref[...], x0b)
            return 0
        jax.lax.fori_loop(0, _B, bstep, 0, unroll=4)
        acc = (_mm(src_ref[...], w_ref[0])
               + _mm(d1a_ref[...], w_ref[1]) + _mm(d2a_ref[...], w_ref[2])
               + _mm(d1b_ref[...], w_ref[3]) + _mm(d2b_ref[...], w_ref[4]))
        del width
        return acc + b_ref[...]

    def dconv_rh(src_ref, w_ref, b_ref, xw):
        def pstep(p, _):
            o1 = p * (2 * _NP)
            sl1 = pl.ds(o1, _NP)
            sl2 = pl.ds(o1 + _NP, _NP)
            rhs = jnp.concatenate(
                [src_ref[sl1, xw:xw + _U], src_ref[sl2, xw:xw + _U]], axis=1)
            x1a = _mm(sa_ref[...], rhs)
            x1b = _mm(sb_ref[...], rhs)
            x2a = _mm(ta_ref[...], rhs)
            x2b = _mm(tb_ref[...], rhs)
            d1a_ref[sl1, xw:xw + _U] = x1a[:, :_U]
            d1a_ref[sl2, xw:xw + _U] = x1a[:, _U:]
            d2a_ref[sl1, xw:xw + _U] = x2a[:, :_U]
            d2a_ref[sl2, xw:xw + _U] = x2a[:, _U:]
            d1b_ref[sl1, xw:xw + _U] = x1b[:, :_U]
            d1b_ref[sl2, xw:xw + _U] = x1b[:, _U:]
            d2b_ref[sl1, xw:xw + _U] = x2b[:, :_U]
            d2b_ref[sl2, xw:xw + _U] = x2b[:, _U:]
            return 0
        jax.lax.fori_loop(0, _B // 2, pstep, 0, unroll=2)
        acc = (_mm(src_ref[...], w_ref[0])
               + _mm(d1a_ref[...], w_ref[1]) + _mm(d2a_ref[...], w_ref[2])
               + _mm(d1b_ref[...], w_ref[3]) + _mm(d2b_ref[...], w_ref[4]))
        return acc + b_ref[...]

    def cell0(x, wg, bg, wc, bc):
        h = h0_ref[...]
        srcA_ref[:, 0:1] = x
        srcA_ref[:, 1:1 + _U] = h
        val = jax.nn.sigmoid(dconv(srcA_ref, wg, bg, 2 * _U))
        r = val[:, :_U]
        u = val[:, _U:]
        srcA_ref[:, 1:1 + _U] = r * h
        c = jnp.tanh(dconv_rh(srcA_ref, wc, bc, 1))
        nh = u * h + (1.0 - u) * c
        h0_ref[...] = nh
        return nh

    def cell1(x, wg, bg, wc, bc):
        h = h1_ref[...]
        srcB_ref[:, :_U] = x
        srcB_ref[:, _U:] = h
        val = jax.nn.sigmoid(dconv(srcB_ref, wg, bg, 2 * _U))
        r = val[:, :_U]
        u = val[:, _U:]
        srcB_ref[:, _U:] = r * h
        c = jnp.tanh(dconv_rh(srcB_ref, wc, bc, _U))
        nh = u * h + (1.0 - u) * c
        h1_ref[...] = nh
        return nh

    h0_ref[...] = jnp.zeros((_R, _U), jnp.float32)
    h1_ref[...] = jnp.zeros((_R, _U), jnp.float32)
    srcA_ref[:, 1 + _U:] = jnp.zeros((_R, _C - 1 - _U), jnp.float32)

    def enc_body(t, carry):
        nh0 = cell0(inp_ref[t], eWg0, ebg0, eWc0, ebc0)
        cell1(nh0, eWg1, ebg1, eWc1, ebc1)
        return carry

    jax.lax.fori_loop(0, _SEQ, enc_body, 0, unroll=False)

    def dec_body(t, dec_in):
        nh0 = cell0(dec_in, dWg0, dbg0, dWc0, dbc0)
        nh1 = cell1(nh0, dWg1, dbg1, dWc1, dbc1)
        proj = _mm(nh1, wproj[...]) + bproj[...]
        out_ref[t] = proj
        return proj

    jax.lax.fori_loop(0, _HOR, dec_body,
                      jnp.zeros((_R, 1), jnp.float32), unroll=False)


def _reorder_w(w, cin, out):
    w = w.reshape(cin, _M, out).transpose(1, 0, 2)
    return jnp.pad(w, ((0, 0), (0, _C - cin), (0, 0)))


def kernel(inputs, adj, params):
    inp = jnp.pad(inputs, ((0, 0), (0, 0), (0, _NP - _N)))
    inp = inp.reshape(_SEQ, _R, 1)
    adj_p = jnp.pad(adj, ((0, _NP - _N), (0, _NP - _N)))

    args = [inp, adj_p]
    for mdl in ("enc", "dec"):
        for l in range(2):
            cin = (1 if l == 0 else _U) + _U
            args.append(_reorder_w(params[f"{mdl}_Wg{l}"], cin, 2 * _U))
            args.append(params[f"{mdl}_bg{l}"].reshape(1, 2 * _U))
            args.append(_reorder_w(params[f"{mdl}_Wc{l}"], cin, _U))
            args.append(params[f"{mdl}_bc{l}"].reshape(1, _U))
    args.append(params["W_proj"])
    args.append(params["b_proj"].reshape(1, 1))

    out = pl.pallas_call(
        _fism_body,
        out_shape=jax.ShapeDtypeStruct((_HOR, _R, 1), jnp.float32),
        scratch_shapes=[pltpu.VMEM((_R, _U), jnp.float32),
                        pltpu.VMEM((_R, _U), jnp.float32),
                        pltpu.VMEM((_NP, _NP), jnp.float32),
                        pltpu.VMEM((_NP, _NP), jnp.float32),
                        pltpu.VMEM((_NP, _NP), jnp.float32),
                        pltpu.VMEM((_NP, _NP), jnp.float32),
                        pltpu.VMEM((_R, _C), jnp.float32),
                        pltpu.VMEM((_R, _C), jnp.float32),
                        pltpu.VMEM((_R, _C), jnp.float32),
                        pltpu.VMEM((_R, _C), jnp.float32),
                        pltpu.VMEM((_R, _C), jnp.float32),
                        pltpu.VMEM((_R, _C), jnp.float32)],
    )(*args)

    return out.reshape(_HOR, _B, _NP)[:, :, :_N]
